# baseline (speedup 1.0000x reference)
import functools

import jax
import jax.numpy as jnp
from jax import lax
from jax.experimental import pallas as pl
from jax.experimental.pallas import tpu as pltpu
from jax.experimental.pallas import tpu_sc as plsc

NUM_FEATURES = 128
VOCAB = 37
N_ATOMS = 100000

_NC = 2
_NS = 16
_NW = _NC * _NS

_CHUNK = 128
_N_FULL = N_ATOMS // _CHUNK
_REM = N_ATOMS - _N_FULL * _CHUNK
_JPW = 25
_SPAN = _JPW * _CHUNK
_VALID_LAST = N_ATOMS - 31 * _SPAN
_NBUF = 6
_LOOK = _NBUF // 2


def _table_body(elec_ref, we_ref, nuc_ref, wls_ref, b_ref, out_ref):
    h = nuc_ref[...] + lax.dot_general(
        elec_ref[...], we_ref[...], (((1,), (1,)), ((), ())),
        preferred_element_type=jnp.float32)
    o = lax.dot_general(
        h, wls_ref[...], (((1,), (1,)), ((), ())),
        preferred_element_type=jnp.float32) + b_ref[...]
    out_ref[...] = o * jax.nn.sigmoid(o)


def _compute_table(elec, W_elec, nuclare_table, W_ls, b_ls):
    return pl.pallas_call(
        _table_body,
        out_shape=jax.ShapeDtypeStruct((VOCAB, NUM_FEATURES), jnp.float32),
    )(elec, W_elec, nuclare_table, W_ls, b_ls.reshape(1, NUM_FEATURES))


_mesh = plsc.VectorSubcoreMesh(core_axis_name="c", subcore_axis_name="s")


@functools.partial(
    pl.kernel,
    mesh=_mesh,
    out_type=jax.ShapeDtypeStruct((N_ATOMS, NUM_FEATURES), jnp.float32),
    scratch_types=[
        pltpu.VMEM((_JPW, _CHUNK), jnp.int32),
        pltpu.VMEM_SHARED((VOCAB, NUM_FEATURES), jnp.float32),
    ] + [pltpu.VMEM((_CHUNK, NUM_FEATURES), jnp.float32)] * _NBUF
      + [pltpu.SemaphoreType.DMA] * (2 * _NBUF),
)
def _sc_gather(table_hbm, z3d_hbm, out_hbm, idx_all, table_sp,
               buf0, buf1, buf2, buf3, buf4, buf5,
               g0, g1, g2, g3, g4, g5, s0, s1, s2, s3, s4, s5):
    wid = lax.axis_index("s") * _NC + lax.axis_index("c")
    bufs = [buf0, buf1, buf2, buf3, buf4, buf5]
    gsems = [g0, g1, g2, g3, g4, g5]
    ssems = [s0, s1, s2, s3, s4, s5]

    @pl.when(lax.axis_index("s") == 0)
    def _():
        pltpu.sync_copy(table_hbm, table_sp)

    plsc.subcore_barrier()

    pltpu.sync_copy(z3d_hbm.at[wid], idx_all)

    def gather_desc(j, b):
        return pltpu.make_async_copy(
            table_sp.at[idx_all.at[j]],
            bufs[b], gsems[b])

    def full_desc(j, b):
        row0 = (wid * _JPW + j) * _CHUNK
        return pltpu.make_async_copy(
            bufs[b], out_hbm.at[pl.ds(row0, _CHUNK)], ssems[b])

    def part_desc(b):
        return pltpu.make_async_copy(
            bufs[b].at[pl.ds(0, _REM)],
            out_hbm.at[pl.ds(_N_FULL * _CHUNK, _REM)], ssems[b])

    def store_op(j, b, op):
        c = wid * _JPW + j

        @pl.when(c < _N_FULL)
        def _():
            op(full_desc(j, b))

        @pl.when(c == _N_FULL)
        def _():
            op(part_desc(b))

    for j in range(_LOOK):
        gather_desc(j, j).start()

    for j in range(_LOOK):
        gather_desc(j + _LOOK, j + _LOOK).start()
        gather_desc(j, j).wait()
        full_desc(j, j).start()

    @pl.loop(0, 3)
    def _steady(gi):
        for b in range(_NBUF):
            j = _LOOK + gi * _NBUF + b
            store_op(j - _LOOK, b, lambda d: d.wait())
            gather_desc(j + _LOOK, b).start()
            jb = (_LOOK + b) % _NBUF
            gather_desc(j, jb).wait()
            store_op(j, jb, lambda d: d.start())

    for j in range(21, 24):
        store_op(j - _LOOK, (j - _LOOK) % _NBUF, lambda d: d.wait())
        if j + _LOOK <= _JPW - 1:
            gather_desc(j + _LOOK, (j + _LOOK) % _NBUF).start()
        gather_desc(j, j % _NBUF).wait()
        store_op(j, j % _NBUF, lambda d: d.start())
    for j in range(21, 24):
        store_op(j, j % _NBUF, lambda d: d.wait())

    gather_desc(_JPW - 1, (_JPW - 1) % _NBUF).wait()
    c_tail = (_JPW - 1) + wid * _JPW

    @pl.when(c_tail < _N_FULL)
    def _():
        pltpu.sync_copy(
            bufs[(_JPW - 1) % _NBUF],
            out_hbm.at[pl.ds(c_tail * _CHUNK, _CHUNK)])


def kernel(z, elec, W_elec, nuclare_table, W_ls, b_ls):
    table = _compute_table(elec, W_elec, nuclare_table, W_ls, b_ls)
    z_pad = jnp.pad(z, (0, _NW * _SPAN - N_ATOMS))
    return _sc_gather(table, z_pad.reshape(_NW, _JPW, _CHUNK))


# speedup vs baseline: 1.0014x; 1.0014x over previous
import functools

import jax
import jax.numpy as jnp
from jax import lax
from jax.experimental import pallas as pl
from jax.experimental.pallas import tpu as pltpu
from jax.experimental.pallas import tpu_sc as plsc

NUM_FEATURES = 128
VOCAB = 37
N_ATOMS = 100000

_NC = 2
_NS = 16
_NW = _NC * _NS

_CHUNK = 128
_N_FULL = N_ATOMS // _CHUNK
_REM = N_ATOMS - _N_FULL * _CHUNK
_JPW = 25
_SPAN = _JPW * _CHUNK
_VALID_LAST = N_ATOMS - 31 * _SPAN
_NBUF = 6
_LOOK = _NBUF // 2


def _table_body(elec_hbm, we_hbm, nuc_hbm, wls_hbm, b_hbm, out_ref,
                elec_v, we_v, nuc_v, wls_v, b_v, sem):
    pltpu.make_async_copy(elec_hbm, elec_v, sem).start()
    pltpu.make_async_copy(we_hbm, we_v, sem).start()
    pltpu.make_async_copy(nuc_hbm, nuc_v, sem).start()
    pltpu.make_async_copy(wls_hbm, wls_v, sem).start()
    pltpu.make_async_copy(b_hbm, b_v, sem).start()
    pltpu.make_async_copy(elec_hbm, elec_v, sem).wait()
    pltpu.make_async_copy(we_hbm, we_v, sem).wait()
    pltpu.make_async_copy(nuc_hbm, nuc_v, sem).wait()
    pltpu.make_async_copy(wls_hbm, wls_v, sem).wait()
    pltpu.make_async_copy(b_hbm, b_v, sem).wait()
    h = nuc_v[...] + lax.dot_general(
        elec_v[...], we_v[...], (((1,), (1,)), ((), ())),
        preferred_element_type=jnp.float32)
    o = lax.dot_general(
        h, wls_v[...], (((1,), (1,)), ((), ())),
        preferred_element_type=jnp.float32) + b_v[...]
    out_ref[...] = o * jax.nn.sigmoid(o)


def _compute_table(elec, W_elec, nuclare_table, W_ls, b_ls):
    any_spec = pl.BlockSpec(memory_space=pl.ANY)
    return pl.pallas_call(
        _table_body,
        in_specs=[any_spec] * 5,
        out_shape=jax.ShapeDtypeStruct((VOCAB, NUM_FEATURES), jnp.float32),
        scratch_shapes=[
            pltpu.VMEM((VOCAB, 16), jnp.float32),
            pltpu.VMEM((NUM_FEATURES, 16), jnp.float32),
            pltpu.VMEM((VOCAB, NUM_FEATURES), jnp.float32),
            pltpu.VMEM((NUM_FEATURES, NUM_FEATURES), jnp.float32),
            pltpu.VMEM((1, NUM_FEATURES), jnp.float32),
            pltpu.SemaphoreType.DMA,
        ],
    )(elec, W_elec, nuclare_table, W_ls, b_ls.reshape(1, NUM_FEATURES))


_mesh = plsc.VectorSubcoreMesh(core_axis_name="c", subcore_axis_name="s")


@functools.partial(
    pl.kernel,
    mesh=_mesh,
    out_type=jax.ShapeDtypeStruct((N_ATOMS, NUM_FEATURES), jnp.float32),
    scratch_types=[
        pltpu.VMEM((_JPW, _CHUNK), jnp.int32),
        pltpu.VMEM_SHARED((VOCAB, NUM_FEATURES), jnp.float32),
    ] + [pltpu.VMEM((_CHUNK, NUM_FEATURES), jnp.float32)] * _NBUF
      + [pltpu.SemaphoreType.DMA] * (2 * _NBUF),
)
def _sc_gather(table_hbm, z3d_hbm, out_hbm, idx_all, table_sp,
               buf0, buf1, buf2, buf3, buf4, buf5,
               g0, g1, g2, g3, g4, g5, s0, s1, s2, s3, s4, s5):
    wid = lax.axis_index("s") * _NC + lax.axis_index("c")
    bufs = [buf0, buf1, buf2, buf3, buf4, buf5]
    gsems = [g0, g1, g2, g3, g4, g5]
    ssems = [s0, s1, s2, s3, s4, s5]

    @pl.when(lax.axis_index("s") == 0)
    def _():
        pltpu.sync_copy(table_hbm, table_sp)

    plsc.subcore_barrier()

    pltpu.sync_copy(z3d_hbm.at[wid], idx_all)

    def gather_desc(j, b):
        return pltpu.make_async_copy(
            table_sp.at[idx_all.at[j]],
            bufs[b], gsems[b])

    def full_desc(j, b):
        row0 = (wid * _JPW + j) * _CHUNK
        return pltpu.make_async_copy(
            bufs[b], out_hbm.at[pl.ds(row0, _CHUNK)], ssems[b])

    def part_desc(b):
        return pltpu.make_async_copy(
            bufs[b].at[pl.ds(0, _REM)],
            out_hbm.at[pl.ds(_N_FULL * _CHUNK, _REM)], ssems[b])

    def store_op(j, b, op):
        c = wid * _JPW + j

        @pl.when(c < _N_FULL)
        def _():
            op(full_desc(j, b))

        @pl.when(c == _N_FULL)
        def _():
            op(part_desc(b))

    for j in range(_LOOK):
        gather_desc(j, j).start()

    for j in range(_LOOK):
        gather_desc(j + _LOOK, j + _LOOK).start()
        gather_desc(j, j).wait()
        full_desc(j, j).start()

    @pl.loop(0, 3)
    def _steady(gi):
        for b in range(_NBUF):
            j = _LOOK + gi * _NBUF + b
            store_op(j - _LOOK, b, lambda d: d.wait())
            gather_desc(j + _LOOK, b).start()
            jb = (_LOOK + b) % _NBUF
            gather_desc(j, jb).wait()
            store_op(j, jb, lambda d: d.start())

    for j in range(21, 24):
        store_op(j - _LOOK, (j - _LOOK) % _NBUF, lambda d: d.wait())
        if j + _LOOK <= _JPW - 1:
            gather_desc(j + _LOOK, (j + _LOOK) % _NBUF).start()
        gather_desc(j, j % _NBUF).wait()
        store_op(j, j % _NBUF, lambda d: d.start())
    for j in range(21, 24):
        store_op(j, j % _NBUF, lambda d: d.wait())

    gather_desc(_JPW - 1, (_JPW - 1) % _NBUF).wait()
    c_tail = (_JPW - 1) + wid * _JPW

    @pl.when(c_tail < _N_FULL)
    def _():
        pltpu.sync_copy(
            bufs[(_JPW - 1) % _NBUF],
            out_hbm.at[pl.ds(c_tail * _CHUNK, _CHUNK)])


def kernel(z, elec, W_elec, nuclare_table, W_ls, b_ls):
    table = _compute_table(elec, W_elec, nuclare_table, W_ls, b_ls)
    z_pad = jnp.pad(z, (0, _NW * _SPAN - N_ATOMS))
    return _sc_gather(table, z_pad.reshape(_NW, _JPW, _CHUNK))


# speedup vs baseline: 1.0153x; 1.0139x over previous
import functools

import jax
import jax.numpy as jnp
from jax import lax
from jax.experimental import pallas as pl
from jax.experimental.pallas import tpu as pltpu
from jax.experimental.pallas import tpu_sc as plsc

NUM_FEATURES = 128
VOCAB = 37
N_ATOMS = 100000

_NC = 2
_NS = 16
_NW = _NC * _NS

_CHUNK = 128
_N_FULL = N_ATOMS // _CHUNK
_REM = N_ATOMS - _N_FULL * _CHUNK
_JPW = 25
_SPAN = _JPW * _CHUNK
_VALID_LAST = N_ATOMS - 31 * _SPAN
_NBUF = 6
_LOOK = _NBUF // 2


def _table_body(elec_hbm, we_hbm, nuc_hbm, wls_hbm, b_hbm, out_ref,
                elec_v, we_v, nuc_v, wls_v, b_v, sem):
    pltpu.make_async_copy(elec_hbm, elec_v, sem).start()
    pltpu.make_async_copy(we_hbm, we_v, sem).start()
    pltpu.make_async_copy(nuc_hbm, nuc_v, sem).start()
    pltpu.make_async_copy(wls_hbm, wls_v, sem).start()
    pltpu.make_async_copy(b_hbm, b_v, sem).start()
    pltpu.make_async_copy(elec_hbm, elec_v, sem).wait()
    pltpu.make_async_copy(we_hbm, we_v, sem).wait()
    pltpu.make_async_copy(nuc_hbm, nuc_v, sem).wait()
    pltpu.make_async_copy(wls_hbm, wls_v, sem).wait()
    pltpu.make_async_copy(b_hbm, b_v, sem).wait()
    h = nuc_v[...] + lax.dot_general(
        elec_v[...], we_v[...], (((1,), (1,)), ((), ())),
        preferred_element_type=jnp.float32)
    o = lax.dot_general(
        h, wls_v[...], (((1,), (1,)), ((), ())),
        preferred_element_type=jnp.float32) + b_v[...]
    out_ref[...] = o * jax.nn.sigmoid(o)


def _compute_table(elec, W_elec, nuclare_table, W_ls, b_ls):
    any_spec = pl.BlockSpec(memory_space=pl.ANY)
    return pl.pallas_call(
        _table_body,
        in_specs=[any_spec] * 5,
        out_shape=jax.ShapeDtypeStruct((VOCAB, NUM_FEATURES), jnp.float32),
        scratch_shapes=[
            pltpu.VMEM((VOCAB, 16), jnp.float32),
            pltpu.VMEM((NUM_FEATURES, 16), jnp.float32),
            pltpu.VMEM((VOCAB, NUM_FEATURES), jnp.float32),
            pltpu.VMEM((NUM_FEATURES, NUM_FEATURES), jnp.float32),
            pltpu.VMEM((1, NUM_FEATURES), jnp.float32),
            pltpu.SemaphoreType.DMA,
        ],
    )(elec, W_elec, nuclare_table, W_ls, b_ls.reshape(1, NUM_FEATURES))


_mesh = plsc.VectorSubcoreMesh(core_axis_name="c", subcore_axis_name="s")


@functools.partial(
    pl.kernel,
    mesh=_mesh,
    out_type=jax.ShapeDtypeStruct((N_ATOMS, NUM_FEATURES), jnp.float32),
    scratch_types=[
        pltpu.VMEM((_JPW, _CHUNK), jnp.int32),
        pltpu.VMEM_SHARED((VOCAB, NUM_FEATURES), jnp.float32),
    ] + [pltpu.VMEM((_CHUNK, NUM_FEATURES), jnp.float32)] * _NBUF
      + [pltpu.SemaphoreType.DMA] * (2 * _NBUF),
)
def _sc_gather(table_hbm, z3d_hbm, out_hbm, idx_all, table_sp,
               buf0, buf1, buf2, buf3, buf4, buf5,
               g0, g1, g2, g3, g4, g5, s0, s1, s2, s3, s4, s5):
    wid = lax.axis_index("s") * _NC + lax.axis_index("c")
    bufs = [buf0, buf1, buf2, buf3, buf4, buf5]
    gsems = [g0, g1, g2, g3, g4, g5]
    ssems = [s0, s1, s2, s3, s4, s5]

    idx_stage = pltpu.make_async_copy(z3d_hbm.at[wid], idx_all, ssems[5])
    idx_stage.start()

    @pl.when(lax.axis_index("s") == 0)
    def _():
        pltpu.sync_copy(table_hbm, table_sp)

    plsc.subcore_barrier()
    idx_stage.wait()

    def gather_desc(j, b):
        return pltpu.make_async_copy(
            table_sp.at[idx_all.at[j]],
            bufs[b], gsems[b])

    def full_desc(j, b):
        row0 = (wid * _JPW + j) * _CHUNK
        return pltpu.make_async_copy(
            bufs[b], out_hbm.at[pl.ds(row0, _CHUNK)], ssems[b])

    def part_desc(b):
        return pltpu.make_async_copy(
            bufs[b].at[pl.ds(0, _REM)],
            out_hbm.at[pl.ds(_N_FULL * _CHUNK, _REM)], ssems[b])

    def store_op(j, b, op):
        c = wid * _JPW + j

        @pl.when(c < _N_FULL)
        def _():
            op(full_desc(j, b))

        @pl.when(c == _N_FULL)
        def _():
            op(part_desc(b))

    for j in range(_LOOK):
        gather_desc(j, j).start()

    for j in range(_LOOK):
        gather_desc(j + _LOOK, j + _LOOK).start()
        gather_desc(j, j).wait()
        full_desc(j, j).start()

    @pl.loop(0, 3)
    def _steady(gi):
        for b in range(_NBUF):
            j = _LOOK + gi * _NBUF + b
            store_op(j - _LOOK, b, lambda d: d.wait())
            gather_desc(j + _LOOK, b).start()
            jb = (_LOOK + b) % _NBUF
            gather_desc(j, jb).wait()
            store_op(j, jb, lambda d: d.start())

    for j in range(21, 24):
        store_op(j - _LOOK, (j - _LOOK) % _NBUF, lambda d: d.wait())
        if j + _LOOK <= _JPW - 1:
            gather_desc(j + _LOOK, (j + _LOOK) % _NBUF).start()
        gather_desc(j, j % _NBUF).wait()
        store_op(j, j % _NBUF, lambda d: d.start())
    for j in range(21, 24):
        store_op(j, j % _NBUF, lambda d: d.wait())

    gather_desc(_JPW - 1, (_JPW - 1) % _NBUF).wait()
    c_tail = (_JPW - 1) + wid * _JPW

    @pl.when(c_tail < _N_FULL)
    def _():
        pltpu.sync_copy(
            bufs[(_JPW - 1) % _NBUF],
            out_hbm.at[pl.ds(c_tail * _CHUNK, _CHUNK)])


def kernel(z, elec, W_elec, nuclare_table, W_ls, b_ls):
    table = _compute_table(elec, W_elec, nuclare_table, W_ls, b_ls)
    z_pad = jnp.pad(z, (0, _NW * _SPAN - N_ATOMS))
    return _sc_gather(table, z_pad.reshape(_NW, _JPW, _CHUNK))


# speedup vs baseline: 1.0875x; 1.0711x over previous
import functools

import jax
import jax.numpy as jnp
from jax import lax
from jax.experimental import pallas as pl
from jax.experimental.pallas import tpu as pltpu
from jax.experimental.pallas import tpu_sc as plsc

NUM_FEATURES = 128
VOCAB = 37
N_ATOMS = 100000

_NC = 2
_NS = 16
_NW = _NC * _NS

_CHUNK = 128
_N_FULL = N_ATOMS // _CHUNK
_REM = N_ATOMS - _N_FULL * _CHUNK
_JPW = 25
_SPAN = _JPW * _CHUNK
_VALID_LAST = N_ATOMS - 31 * _SPAN
_NBUF = 6
_LOOK = _NBUF // 2


def _table_body(elec_hbm, we_hbm, nuc_hbm, wls_hbm, b_hbm, out_ref,
                elec_v, we_v, nuc_v, wls_v, b_v, sem):
    pltpu.make_async_copy(elec_hbm, elec_v, sem).start()
    pltpu.make_async_copy(we_hbm, we_v, sem).start()
    pltpu.make_async_copy(nuc_hbm, nuc_v, sem).start()
    pltpu.make_async_copy(wls_hbm, wls_v, sem).start()
    pltpu.make_async_copy(b_hbm, b_v, sem).start()
    pltpu.make_async_copy(elec_hbm, elec_v, sem).wait()
    pltpu.make_async_copy(we_hbm, we_v, sem).wait()
    pltpu.make_async_copy(nuc_hbm, nuc_v, sem).wait()
    pltpu.make_async_copy(wls_hbm, wls_v, sem).wait()
    pltpu.make_async_copy(b_hbm, b_v, sem).wait()
    h = nuc_v[...] + lax.dot_general(
        elec_v[...], we_v[...], (((0,), (0,)), ((), ())),
        preferred_element_type=jnp.float32)
    o = lax.dot_general(
        h, wls_v[...], (((1,), (1,)), ((), ())),
        preferred_element_type=jnp.float32) + b_v[...]
    out_ref[...] = o * jax.nn.sigmoid(o)


def _compute_table(elec, W_elec, nuclare_table, W_ls, b_ls):
    any_spec = pl.BlockSpec(memory_space=pl.ANY)
    return pl.pallas_call(
        _table_body,
        in_specs=[any_spec] * 5,
        out_shape=jax.ShapeDtypeStruct((VOCAB, NUM_FEATURES), jnp.float32),
        scratch_shapes=[
            pltpu.VMEM((16, VOCAB), jnp.float32),
            pltpu.VMEM((16, NUM_FEATURES), jnp.float32),
            pltpu.VMEM((VOCAB, NUM_FEATURES), jnp.float32),
            pltpu.VMEM((NUM_FEATURES, NUM_FEATURES), jnp.float32),
            pltpu.VMEM((1, NUM_FEATURES), jnp.float32),
            pltpu.SemaphoreType.DMA,
        ],
    )(elec.T, W_elec.T, nuclare_table, W_ls, b_ls.reshape(1, NUM_FEATURES))


_mesh = plsc.VectorSubcoreMesh(core_axis_name="c", subcore_axis_name="s")


@functools.partial(
    pl.kernel,
    mesh=_mesh,
    out_type=jax.ShapeDtypeStruct((N_ATOMS, NUM_FEATURES), jnp.float32),
    scratch_types=[
        pltpu.VMEM((_JPW, _CHUNK), jnp.int32),
        pltpu.VMEM_SHARED((VOCAB, NUM_FEATURES), jnp.float32),
    ] + [pltpu.VMEM((_CHUNK, NUM_FEATURES), jnp.float32)] * _NBUF
      + [pltpu.SemaphoreType.DMA] * (2 * _NBUF),
)
def _sc_gather(table_hbm, z3d_hbm, out_hbm, idx_all, table_sp,
               buf0, buf1, buf2, buf3, buf4, buf5,
               g0, g1, g2, g3, g4, g5, s0, s1, s2, s3, s4, s5):
    wid = lax.axis_index("s") * _NC + lax.axis_index("c")
    bufs = [buf0, buf1, buf2, buf3, buf4, buf5]
    gsems = [g0, g1, g2, g3, g4, g5]
    ssems = [s0, s1, s2, s3, s4, s5]

    idx_stage = pltpu.make_async_copy(z3d_hbm.at[wid], idx_all, ssems[5])
    idx_stage.start()

    @pl.when(lax.axis_index("s") == 0)
    def _():
        pltpu.sync_copy(table_hbm, table_sp)

    plsc.subcore_barrier()
    idx_stage.wait()

    def gather_desc(j, b):
        return pltpu.make_async_copy(
            table_sp.at[idx_all.at[j]],
            bufs[b], gsems[b])

    def full_desc(j, b):
        row0 = (wid * _JPW + j) * _CHUNK
        return pltpu.make_async_copy(
            bufs[b], out_hbm.at[pl.ds(row0, _CHUNK)], ssems[b])

    def part_desc(b):
        return pltpu.make_async_copy(
            bufs[b].at[pl.ds(0, _REM)],
            out_hbm.at[pl.ds(_N_FULL * _CHUNK, _REM)], ssems[b])

    def store_op(j, b, op):
        c = wid * _JPW + j

        @pl.when(c < _N_FULL)
        def _():
            op(full_desc(j, b))

        @pl.when(c == _N_FULL)
        def _():
            op(part_desc(b))

    for j in range(_LOOK):
        gather_desc(j, j).start()

    for j in range(_LOOK):
        gather_desc(j + _LOOK, j + _LOOK).start()
        gather_desc(j, j).wait()
        full_desc(j, j).start()

    @pl.loop(0, 3)
    def _steady(gi):
        for b in range(_NBUF):
            j = _LOOK + gi * _NBUF + b
            store_op(j - _LOOK, b, lambda d: d.wait())
            gather_desc(j + _LOOK, b).start()
            jb = (_LOOK + b) % _NBUF
            gather_desc(j, jb).wait()
            store_op(j, jb, lambda d: d.start())

    for j in range(21, 24):
        store_op(j - _LOOK, (j - _LOOK) % _NBUF, lambda d: d.wait())
        if j + _LOOK <= _JPW - 1:
            gather_desc(j + _LOOK, (j + _LOOK) % _NBUF).start()
        gather_desc(j, j % _NBUF).wait()
        store_op(j, j % _NBUF, lambda d: d.start())
    for j in range(21, 24):
        store_op(j, j % _NBUF, lambda d: d.wait())

    gather_desc(_JPW - 1, (_JPW - 1) % _NBUF).wait()
    c_tail = (_JPW - 1) + wid * _JPW

    @pl.when(c_tail < _N_FULL)
    def _():
        pltpu.sync_copy(
            bufs[(_JPW - 1) % _NBUF],
            out_hbm.at[pl.ds(c_tail * _CHUNK, _CHUNK)])


def kernel(z, elec, W_elec, nuclare_table, W_ls, b_ls):
    table = _compute_table(elec, W_elec, nuclare_table, W_ls, b_ls)
    z_pad = jnp.pad(z, (0, _NW * _SPAN - N_ATOMS))
    return _sc_gather(table, z_pad.reshape(_NW, _JPW, _CHUNK))
